# baseline (speedup 1.0000x reference)
import jax
import jax.numpy as jnp
from jax.experimental import pallas as pl
from jax.experimental.pallas import tpu as pltpu

_K = 64
_R = 32
_N = 32768
_W = 512
_NS = _N // _W


def _enc(f):
    bi = jax.lax.bitcast_convert_type(f, jnp.int32)
    return jnp.where(bi >= 0, bi, jnp.int32(-2147483648) - bi)


def _dec(e):
    bi = jnp.where(e >= 0, e, jnp.int32(-2147483648) - e)
    return jax.lax.bitcast_convert_type(bi, jnp.float32)


def _pass(ref, cand, ns):
    inf = jnp.float32(jnp.inf)
    xs = ref[:, 0:_W]
    km = xs >= cand
    acc_c = km.astype(jnp.int32)
    acc_mn = jnp.where(km, xs, inf)
    acc_mx = jnp.where(km, -inf, xs)
    for k in range(1, ns):
        xs = ref[:, k * _W:(k + 1) * _W]
        km = xs >= cand
        acc_c = acc_c + km.astype(jnp.int32)
        acc_mn = jnp.minimum(acc_mn, jnp.where(km, xs, inf))
        acc_mx = jnp.maximum(acc_mx, jnp.where(km, -inf, xs))
    c = jnp.sum(acc_c, axis=1, keepdims=True)
    smin = jnp.min(acc_mn, axis=1, keepdims=True)
    mlt = jnp.max(acc_mx, axis=1, keepdims=True)
    return c, smin, mlt


def _refine(ref, ns, lo, hi, clo, v1, l1, v0, l0, maxit):
    def _open(lo, hi, clo):
        return (_enc(hi) - _enc(lo) > 1) & (clo != _K)

    def cond(st):
        i, lo, hi, clo, v1, l1, v0, l0 = st
        return (i < maxit) & jnp.any(_open(lo, hi, clo))

    def body(st):
        i, lo, hi, clo, v1, l1, v0, l0 = st
        is_open = _open(lo, hi, clo)
        el, eh = _enc(lo), _enc(hi)
        denom = l0 - l1
        degen = ((jnp.abs(denom) < 1e-6) | (v0 == v1)
                 | (jax.lax.rem(i, jnp.int32(3)) == 2))
        cand_sec = v1 + (6.0 - l1) * (v0 - v1) / jnp.where(degen, 1.0, denom)
        ce = jnp.where(degen, el + (eh - el) // 2, _enc(cand_sec))
        ce = jnp.minimum(jnp.maximum(ce, el + 1), eh - 1)
        cand = _dec(ce)

        c, smin, mlt = _pass(ref, cand, ns)
        lc = jnp.log2(jnp.maximum(c.astype(jnp.float32), 0.5))
        ge = is_open & (c >= _K)
        lt = is_open & (c < _K)
        lo = jnp.where(ge, smin, lo)
        clo = jnp.where(ge, c, clo)
        hi = jnp.where(lt, _dec(_enc(mlt) + 1), hi)
        newv = jnp.where(ge, smin, mlt)
        newl = jnp.where(ge, lc, jnp.log2((c + 1).astype(jnp.float32)))
        v0 = jnp.where(is_open, v1, v0)
        l0 = jnp.where(is_open, l1, l0)
        v1 = jnp.where(is_open, newv, v1)
        l1 = jnp.where(is_open, newl, l1)
        return i + 1, lo, hi, clo, v1, l1, v0, l0

    st = jax.lax.while_loop(cond, body,
                            (jnp.int32(0), lo, hi, clo, v1, l1, v0, l0))
    return st[1:]


def _topk_mask_block(x_ref, o_ref, m_ref):
    m = x_ref[:, 0:_W]
    for k in range(1, _NS):
        m = jnp.maximum(m, x_ref[:, k * _W:(k + 1) * _W])
    m_ref[...] = m
    g = m[:, 0:64]
    for k in range(1, 8):
        g = jnp.maximum(g, m[:, k * 64:(k + 1) * 64])
    lo0 = jnp.min(g, axis=1, keepdims=True)
    hi0 = _dec(_enc(jnp.max(g, axis=1, keepdims=True)) + 1)

    cm0, smin_m, _ = _pass(m_ref, lo0, 1)
    lm1 = jnp.log2(cm0.astype(jnp.float32))
    lm0 = jnp.full((_R, 1), -1.0, dtype=jnp.float32)
    m64, _, _, s1v1, s1l1, s1v0, s1l0 = _refine(
        m_ref, 1, smin_m, hi0, cm0, smin_m, lm1, hi0, lm0, 12)

    c0, smin0, _ = _pass(x_ref, m64, _NS)
    l1 = jnp.log2(c0.astype(jnp.float32))
    t, _, _, _, _, _, _ = _refine(
        x_ref, _NS, smin0, hi0, c0, smin0, l1, s1v0, s1l0, 18)

    x = x_ref[...]
    o_ref[...] = jnp.where(x >= t, x, jnp.float32(0.0))


def kernel(x):
    rows, cols = x.shape
    grid = rows // _R
    return pl.pallas_call(
        _topk_mask_block,
        grid=(grid,),
        in_specs=[pl.BlockSpec((_R, cols), lambda i: (i, 0))],
        out_specs=pl.BlockSpec((_R, cols), lambda i: (i, 0)),
        out_shape=jax.ShapeDtypeStruct(x.shape, x.dtype),
        scratch_shapes=[pltpu.VMEM((_R, _W), jnp.float32)],
    )(x)
